# baseline (speedup 1.0000x reference)
import jax
import jax.numpy as jnp
from jax import lax
from jax.experimental import pallas as pl
from jax.experimental.pallas import tpu as pltpu
from jax.experimental.pallas import tpu_sc as plsc

_B = 16384
_EMBED = 64
_N_CONT = 13
_N_DISC = 26
_SLOTS = _N_CONT + _N_DISC

_NC = 2
_NS = 16
_NW = _NC * _NS
_ROWS_W = _B // _NW
_NB = 8
_CH = _ROWS_W // _NB
_GD = _NB * _N_DISC
_GC = _NB * _N_CONT
_C_PAD = 16


def _sc_body(cont_h, disc_h, dstd_h, dstc_h, wb_h, table_h, out_h,
             idx_v, dstd_v, dstc_v, cont_v, wb_v,
             gbuf0, gbuf1, cbuf0, cbuf1, sem_g, sem_s0, sem_s1):
    c = lax.axis_index("c")
    s = lax.axis_index("s")
    wid = s * _NC + c

    pltpu.sync_copy(disc_h.at[pl.ds(wid * 2 * _CH, 2 * _CH)], idx_v)
    pltpu.sync_copy(dstd_h.at[pl.ds(wid * 2 * _CH, 2 * _CH)], dstd_v)
    pltpu.sync_copy(dstc_h.at[pl.ds(wid * _CH, _CH)], dstc_v)
    pltpu.sync_copy(cont_h.at[pl.ds(wid * _ROWS_W, _ROWS_W)], cont_v)
    pltpu.sync_copy(wb_h, wb_v)

    w = [wb_v[e] for e in range(4)]
    bias = [wb_v[4 + e] for e in range(4)]

    gbufs = (gbuf0, gbuf1)
    cbufs = (cbuf0, cbuf1)
    sems = (sem_s0, sem_s1)

    @pl.loop(0, _CH, step=2)
    def _chunk(g0):
        for p in range(2):
            g = g0 + p
            gbuf = gbufs[p]
            cbuf = cbufs[p]
            sem_s = sems[p]

            @pl.when(g0 > 0)
            def _():
                pltpu.make_async_copy(
                    gbuf, out_h.at[pl.ds(0, _GD)], sem_s).wait()
                pltpu.make_async_copy(
                    cbuf, out_h.at[pl.ds(0, _GC)], sem_s).wait()

            cp0 = pltpu.async_copy(
                table_h.at[idx_v.at[2 * g]], gbuf.at[pl.ds(0, _GC)], sem_g)
            cp1 = pltpu.async_copy(
                table_h.at[idx_v.at[2 * g + 1]], gbuf.at[pl.ds(_GC, _GC)],
                sem_g)

            for i in range(_NB):
                cv = cont_v[g * _NB + i]
                for k in range(_N_CONT):
                    cs = cv[k]
                    r = i * _N_CONT + k
                    for e in range(4):
                        cbuf[r, pl.ds(e * 16, 16)] = cs * w[e] + bias[e]

            cp0.wait()
            cp1.wait()

            pltpu.async_copy(
                gbuf.at[pl.ds(0, _GC)], out_h.at[dstd_v.at[2 * g]], sem_s)
            pltpu.async_copy(
                gbuf.at[pl.ds(_GC, _GC)], out_h.at[dstd_v.at[2 * g + 1]],
                sem_s)
            pltpu.async_copy(cbuf, out_h.at[dstc_v.at[g]], sem_s)

    for p in range(2):
        pltpu.make_async_copy(gbufs[p], out_h.at[pl.ds(0, _GD)], sems[p]).wait()
        pltpu.make_async_copy(cbufs[p], out_h.at[pl.ds(0, _GC)], sems[p]).wait()


@jax.jit
def _mixed_input_sc(cont_pad, disc_flat, dstd, dstc, wb, table):
    mesh = plsc.VectorSubcoreMesh(core_axis_name="c", subcore_axis_name="s")
    kfn = pl.kernel(
        _sc_body,
        out_type=jax.ShapeDtypeStruct((_B * _SLOTS, _EMBED), jnp.float32),
        mesh=mesh,
        compiler_params=pltpu.CompilerParams(use_tc_tiling_on_sc=False),
        scratch_types=[
            pltpu.VMEM((2 * _CH, _GC), jnp.int32),
            pltpu.VMEM((2 * _CH, _GC), jnp.int32),
            pltpu.VMEM((_CH, _GC), jnp.int32),
            pltpu.VMEM((_ROWS_W, _C_PAD), jnp.float32),
            pltpu.VMEM((8, 16), jnp.float32),
            pltpu.VMEM((_GD, _EMBED), jnp.float32),
            pltpu.VMEM((_GD, _EMBED), jnp.float32),
            pltpu.VMEM((_GC, _EMBED), jnp.float32),
            pltpu.VMEM((_GC, _EMBED), jnp.float32),
            pltpu.SemaphoreType.DMA,
            pltpu.SemaphoreType.DMA,
            pltpu.SemaphoreType.DMA,
        ],
    )
    return kfn(cont_pad, disc_flat, dstd, dstc, wb, table)


def kernel(Continuous, Discrete, W_cont, b_cont, emb_table):
    cont_pad = jnp.pad(Continuous, ((0, 0), (0, _C_PAD - _N_CONT)))
    disc_flat = Discrete.reshape(_B * _N_DISC // _GC, _GC)
    brow = jnp.arange(_B, dtype=jnp.int32) * _SLOTS
    dstd = (brow[:, None] + _N_CONT
            + jnp.arange(_N_DISC, dtype=jnp.int32)[None, :]
            ).reshape(_B * _N_DISC // _GC, _GC)
    dstc = (brow[:, None] + jnp.arange(_N_CONT, dtype=jnp.int32)[None, :]
            ).reshape(_B * _N_CONT // _GC, _GC)
    wb = jnp.concatenate(
        [W_cont.reshape(4, 16), b_cont.reshape(4, 16)], axis=0)
    out = _mixed_input_sc(cont_pad, disc_flat, dstd, dstc, wb, emb_table)
    return out.reshape(_B, _SLOTS, _EMBED)
